# baseline (speedup 1.0000x reference)
import jax
import jax.numpy as jnp
from jax import lax
from jax.experimental import pallas as pl
from jax.experimental.pallas import tpu as pltpu
from jax.experimental.pallas import tpu_sc as plsc

_N = 2097152
_NW = 32
_RW = _N // _NW
_CH = 1024
_NCH = _RW // _CH
_NG = _CH // 16


def _sc_body(xf, wt_hbm, tokspl_hbm, b_hbm, wspl_hbm, bspl_hbm, of,
             xia, xp, op, ola, fill,
             wt_buf, tokspl_buf, b_buf, wspl_buf, bspl_buf):
    c = lax.axis_index("c")
    s = lax.axis_index("s")
    wid = s * 2 + c

    pltpu.sync_copy(wt_hbm, wt_buf)
    pltpu.sync_copy(tokspl_hbm, tokspl_buf)
    pltpu.sync_copy(b_hbm, b_buf)
    pltpu.sync_copy(wspl_hbm, wspl_buf)
    pltpu.sync_copy(bspl_hbm, bspl_buf)

    iota = lax.iota(jnp.int32, 16)
    rows2 = iota // 8
    cols8 = iota - rows2 * 8
    pad9 = rows2 * 9 + cols8
    iota9 = iota * 9
    iota17 = iota * 17

    crow = b_buf[pl.ds(0, 16)]
    for k in range(8):
        crow = crow + tokspl_buf[pl.ds(16 * k, 16)] * wt_buf[pl.ds(16 * k, 16)]

    @plsc.parallel_loop(0, _CH, unroll=4)
    def _fill_body(r):
        fill[pl.ds(r * 16, 16)] = crow

    xbase = wid * (_RW * 8)
    obase = wid * (_RW * 16)

    def xsrc(ci):
        return xf.at[1, pl.ds(xbase + ci * (_CH * 8), _CH * 8)]

    def odst(ci):
        return of.at[1, pl.ds(obase + ci * (_CH * 16), _CH * 16)]

    def fdst(ci):
        return of.at[0, pl.ds(obase + ci * (_CH * 16), _CH * 16)]

    def repack_in(xi):
        @plsc.parallel_loop(0, _NG, unroll=2)
        def _body(g):
            for u in range(8):
                t = 8 * g + u
                v = xi[pl.ds(t * 16, 16)]
                plsc.store_scatter(xp, [pad9 + t * 18], v)

    def compute():
        for q in range(4):
            js = [4 * q + jj for jj in range(4)]
            wsp = [[wspl_buf[pl.ds((16 * k + j) * 16, 16)] for k in range(8)]
                   for j in js]
            bsp = [bspl_buf[pl.ds(16 * j, 16)] for j in js]

            @plsc.parallel_loop(0, _NG, unroll=2)
            def _gbody(g):
                xb = iota9 + g * 144
                xv = [plsc.load_gather(xp, [xb + k]) for k in range(8)]
                ob = iota17 + g * 272
                for jj in range(4):
                    acc = bsp[jj]
                    for k in range(8):
                        acc = acc + xv[k] * wsp[jj][k]
                    plsc.store_scatter(op, [ob + js[jj]], acc)

    def repack_out(ol):
        @plsc.parallel_loop(0, _NG, unroll=2)
        def _body(g):
            for u in range(16):
                r = g * 16 + u
                ol[pl.ds(r * 16, 16)] = plsc.load_gather(op, [iota + r * 17])

    def chunk_body(ci, carry):
        pltpu.sync_copy(xsrc(ci), xia)
        repack_in(xia)
        compute()
        repack_out(ola)
        pltpu.sync_copy(ola, odst(ci))
        pltpu.sync_copy(fill, fdst(ci))
        return carry

    lax.fori_loop(0, _NCH, chunk_body, 0)


def kernel(x, token, W, b):
    xf = x.reshape(2, _N * 8)
    wt = W.T.reshape(128)
    tokspl = jnp.repeat(token, 16)
    b128 = jnp.concatenate([b, jnp.zeros((112,), jnp.float32)])
    wspl = jnp.repeat(wt, 16)
    bspl = jnp.repeat(b, 16)
    mesh = plsc.VectorSubcoreMesh(core_axis_name="c", subcore_axis_name="s")
    out = pl.kernel(
        _sc_body,
        out_type=jax.ShapeDtypeStruct((2, _N * 16), jnp.float32),
        mesh=mesh,
        compiler_params=pltpu.CompilerParams(needs_layout_passes=False),
        scratch_types=[
            pltpu.VMEM((_CH * 8,), jnp.float32),
            pltpu.VMEM((_CH * 9,), jnp.float32),
            pltpu.VMEM((_CH * 17,), jnp.float32),
            pltpu.VMEM((_CH * 16,), jnp.float32),
            pltpu.VMEM((_CH * 16,), jnp.float32),
            pltpu.VMEM((128,), jnp.float32),
            pltpu.VMEM((128,), jnp.float32),
            pltpu.VMEM((128,), jnp.float32),
            pltpu.VMEM((2048,), jnp.float32),
            pltpu.VMEM((256,), jnp.float32),
        ],
    )(xf, wt, tokspl, b128, wspl, bspl)
    return out.reshape(2, _N, 16)
